# baseline (speedup 1.0000x reference)
import functools

import jax
import jax.numpy as jnp
from jax import lax
from jax.experimental import pallas as pl
from jax.experimental.pallas import tpu as pltpu
from jax.experimental.pallas import tpu_sc as plsc

EMBED_DIM = 64
NUM_CORES = 2
NUM_SUBCORES = 16
NUM_WORKERS = NUM_CORES * NUM_SUBCORES
CHUNK = 128


@functools.partial(jax.jit, static_argnames=())
def _sc_gather(table, idx_t):
    hist, batch = idx_t.shape
    assert batch % (CHUNK * NUM_WORKERS) == 0 or batch == CHUNK * NUM_WORKERS
    n_blocks = batch // CHUNK
    assert n_blocks == NUM_WORKERS
    assert hist % 2 == 0

    mesh = plsc.VectorSubcoreMesh(
        core_axis_name="c", subcore_axis_name="s",
        num_cores=NUM_CORES, num_subcores=NUM_SUBCORES)

    @functools.partial(
        pl.kernel,
        out_type=jax.ShapeDtypeStruct(
            (hist, 8, n_blocks, 8 * CHUNK), jnp.float32),
        mesh=mesh,
        compiler_params=pltpu.CompilerParams(
            use_tc_tiling_on_sc=False, needs_layout_passes=False),
        scratch_types=[
            pltpu.VMEM((hist, CHUNK), jnp.int32),
            pltpu.VMEM((2, CHUNK, EMBED_DIM), jnp.float32),
            pltpu.VMEM((2, 8, 8 * CHUNK), jnp.float32),
            pltpu.SemaphoreType.DMA,
            pltpu.SemaphoreType.DMA,
        ],
    )
    def gather_kernel(table_hbm, idx_hbm, out_hbm, idx_v, rows_v, tiles_v,
                      gsem, wsem):
        w = lax.axis_index("s") * NUM_CORES + lax.axis_index("c")
        pltpu.sync_copy(idx_hbm.at[:, pl.ds(w * CHUNK, CHUNK)], idx_v)

        def start_gather(h, buf):
            pltpu.async_copy(
                table_hbm.at[idx_v.at[h]], rows_v.at[buf], gsem)

        start_gather(0, 0)
        start_gather(1, 1)

        lanes = lax.iota(jnp.int32, 16)

        @pl.loop(0, hist, step=2)
        def _hloop(h0):
            for p in range(2):
                h = h0 + p
                pltpu.make_async_copy(
                    table_hbm.at[idx_v.at[h]], rows_v.at[p], gsem).wait()

                @pl.when(h >= 2)
                def _retire():
                    pltpu.make_async_copy(
                        tiles_v.at[p], out_hbm.at[0, :, w], wsem).wait()

                @pl.loop(0, EMBED_DIM)
                def _dloop(d):
                    r = d // 8
                    dr = d - r * 8
                    col = jnp.zeros((16,), jnp.int32) + d
                    for q in range(CHUNK // 16):
                        vals = plsc.load_gather(
                            rows_v.at[p], [lanes + (16 * q), col])
                        tiles_v[p, r, pl.ds(dr * CHUNK + 16 * q, 16)] = vals

                pltpu.async_copy(
                    tiles_v.at[p], out_hbm.at[h, :, w], wsem)

                @pl.when(h + 2 < hist)
                def _refill():
                    start_gather(h + 2, p)

        for p in range(2):
            pltpu.make_async_copy(
                tiles_v.at[p], out_hbm.at[0, :, w], wsem).wait()

    return gather_kernel(table, idx_t)


def kernel(indices, vectors):
    batch, hist = indices.shape
    idx_t = indices.T.astype(jnp.int32)
    out5 = _sc_gather(vectors, idx_t)
    out = out5.reshape(hist, 8, batch // CHUNK, 8, CHUNK)
    out = out.transpose(2, 4, 0, 1, 3)
    return out.reshape(batch, hist, EMBED_DIM)


# speedup vs baseline: 1.1862x; 1.1862x over previous
import functools

import jax
import jax.numpy as jnp
from jax import lax
from jax.experimental import pallas as pl
from jax.experimental.pallas import tpu as pltpu
from jax.experimental.pallas import tpu_sc as plsc

EMBED_DIM = 64
NUM_CORES = 2
NUM_SUBCORES = 16
NUM_WORKERS = NUM_CORES * NUM_SUBCORES
CHUNK = 128


@functools.partial(jax.jit, static_argnames=())
def _sc_gather(table, idx_t):
    hist, batch = idx_t.shape
    assert batch % (CHUNK * NUM_WORKERS) == 0 or batch == CHUNK * NUM_WORKERS
    n_blocks = batch // CHUNK
    assert n_blocks == NUM_WORKERS
    assert hist % 2 == 0

    mesh = plsc.VectorSubcoreMesh(
        core_axis_name="c", subcore_axis_name="s",
        num_cores=NUM_CORES, num_subcores=NUM_SUBCORES)

    @functools.partial(
        pl.kernel,
        out_type=jax.ShapeDtypeStruct(
            (hist, 8, n_blocks, 8 * CHUNK), jnp.float32),
        mesh=mesh,
        compiler_params=pltpu.CompilerParams(
            use_tc_tiling_on_sc=False, needs_layout_passes=False),
        scratch_types=[
            pltpu.VMEM((hist, CHUNK), jnp.int32),
            pltpu.VMEM((2, CHUNK, EMBED_DIM), jnp.float32),
            pltpu.VMEM((2, 8, 8 * CHUNK), jnp.float32),
            pltpu.SemaphoreType.DMA,
            pltpu.SemaphoreType.DMA,
        ],
    )
    def gather_kernel(table_hbm, idx_hbm, out_hbm, idx_v, rows_v, tiles_v,
                      gsem, wsem):
        w = lax.axis_index("s") * NUM_CORES + lax.axis_index("c")
        pltpu.sync_copy(idx_hbm.at[:, pl.ds(w * CHUNK, CHUNK)], idx_v)

        def start_gather(h, buf):
            pltpu.async_copy(
                table_hbm.at[idx_v.at[h]], rows_v.at[buf], gsem)

        start_gather(0, 0)
        start_gather(1, 1)

        lanes = lax.iota(jnp.int32, 16)

        @pl.loop(0, hist, step=2)
        def _hloop(h0):
            for p in range(2):
                h = h0 + p
                pltpu.make_async_copy(
                    table_hbm.at[idx_v.at[h]], rows_v.at[p], gsem).wait()

                @pl.when(h >= 2)
                def _retire():
                    pltpu.make_async_copy(
                        tiles_v.at[p], out_hbm.at[0, :, w], wsem).wait()

                @plsc.parallel_loop(0, EMBED_DIM, unroll=2)
                def _dloop(d):
                    r = d // 8
                    dr = d - r * 8
                    col = jnp.zeros((16,), jnp.int32) + d
                    for q in range(CHUNK // 16):
                        vals = plsc.load_gather(
                            rows_v.at[p], [lanes + (16 * q), col])
                        tiles_v[p, r, pl.ds(dr * CHUNK + 16 * q, 16)] = vals

                pltpu.async_copy(
                    tiles_v.at[p], out_hbm.at[h, :, w], wsem)

                @pl.when(h + 2 < hist)
                def _refill():
                    start_gather(h + 2, p)

        for p in range(2):
            pltpu.make_async_copy(
                tiles_v.at[p], out_hbm.at[0, :, w], wsem).wait()

    return gather_kernel(table, idx_t)


def kernel(indices, vectors):
    batch, hist = indices.shape
    idx_t = indices.T.astype(jnp.int32)
    out5 = _sc_gather(vectors, idx_t)
    out = out5.reshape(hist, 8, batch // CHUNK, 8, CHUNK)
    out = out.transpose(2, 4, 0, 1, 3)
    return out.reshape(batch, hist, EMBED_DIM)
